# Initial kernel scaffold; baseline (speedup 1.0000x reference)
import jax
import jax.numpy as jnp
from jax.experimental import pallas as pl


def kernel(x, edge_index, edge_features, Wn, bn, We, be, Wm, bm, Wlin, blin):
    raise NotImplementedError("write your pallas kernel here")


# speedup vs baseline: 2.4642x; 2.4642x over previous
import functools

import jax
import jax.numpy as jnp
from jax import lax
from jax.experimental import pallas as pl
from jax.experimental.pallas import tpu as pltpu
from jax.experimental.pallas import tpu_sc as plsc

_N = 10000
_E = 320000
_D = 128
_DE = 16
_L = 4
_EMB = 128

_NC = 2
_NS = 16
_NW = _NC * _NS
_CH = 80
_RPW = _E // _NW
_NCH = _RPW // _CH
_TPS = 624
_TAIL = _N - _NS * _TPS

_RB = 2000
_BE = 2000


def _node0_body(h_ref, w_ref, b_ref, o_ref):
    o_ref[...] = (
        jnp.dot(h_ref[...], w_ref[...], preferred_element_type=jnp.float32)
        + b_ref[...]
    )


def _tc_node0(h, w, b):
    return pl.pallas_call(
        _node0_body,
        grid=(_N // _RB,),
        in_specs=[
            pl.BlockSpec((_RB, _D), lambda i: (i, 0)),
            pl.BlockSpec((_D, _D), lambda i: (0, 0)),
            pl.BlockSpec((1, _D), lambda i: (0, 0)),
        ],
        out_specs=pl.BlockSpec((_RB, _D), lambda i: (i, 0)),
        out_shape=jax.ShapeDtypeStruct((_N, _D), jnp.float32),
    )(h, w, b)


def _node_upd_body(hn_ref, pa_ref, pb_ref, w_ref, b_ref, o_ref):
    h = hn_ref[...] + pa_ref[...] + pb_ref[...]
    o_ref[...] = (
        jnp.dot(h, w_ref[...], preferred_element_type=jnp.float32) + b_ref[...]
    )


def _tc_node_upd(hn, pa, pb, w, b):
    return pl.pallas_call(
        _node_upd_body,
        grid=(_N // _RB,),
        in_specs=[
            pl.BlockSpec((_RB, _D), lambda i: (i, 0)),
            pl.BlockSpec((_RB, _D), lambda i: (i, 0)),
            pl.BlockSpec((_RB, _D), lambda i: (i, 0)),
            pl.BlockSpec((_D, _D), lambda i: (0, 0)),
            pl.BlockSpec((1, _D), lambda i: (0, 0)),
        ],
        out_specs=pl.BlockSpec((_RB, _D), lambda i: (i, 0)),
        out_shape=jax.ShapeDtypeStruct((_N, _D), jnp.float32),
    )(hn, pa, pb, w, b)


def _edge_body(g_ref, ef_ref, we_ref, be_ref, wm_ref, bm_ref, o_ref):
    me = (
        jnp.dot(ef_ref[...], we_ref[...], preferred_element_type=jnp.float32)
        + be_ref[...]
    )
    t = g_ref[...] * me
    o_ref[...] = jnp.tanh(
        jnp.dot(t, wm_ref[...], preferred_element_type=jnp.float32) + bm_ref[...]
    )


def _tc_edge(g, ef, we, be, wm, bm):
    return pl.pallas_call(
        _edge_body,
        grid=(_E // _BE,),
        in_specs=[
            pl.BlockSpec((_BE, _D), lambda i: (i, 0)),
            pl.BlockSpec((_BE, _DE), lambda i: (i, 0)),
            pl.BlockSpec((_DE, _D), lambda i: (0, 0)),
            pl.BlockSpec((1, _D), lambda i: (0, 0)),
            pl.BlockSpec((_D, _D), lambda i: (0, 0)),
            pl.BlockSpec((1, _D), lambda i: (0, 0)),
        ],
        out_specs=pl.BlockSpec((_BE, _D), lambda i: (i, 0)),
        out_shape=jax.ShapeDtypeStruct((_E, _D), jnp.float32),
    )(g, ef, we, be, wm, bm)


def _final_body(hn_ref, pa_ref, pb_ref, wl_ref, bl_ref, o_ref):
    h = hn_ref[...] + pa_ref[...] + pb_ref[...]
    s = jnp.sum(h, axis=0, keepdims=True) * (1.0 / _N)
    o_ref[...] = (
        jnp.dot(s, wl_ref[...], preferred_element_type=jnp.float32) + bl_ref[...]
    )


def _tc_final(hn, pa, pb, wl, bl):
    return pl.pallas_call(
        _final_body,
        out_shape=jax.ShapeDtypeStruct((1, _EMB), jnp.float32),
    )(hn, pa, pb, wl, bl)


def _sc_gather(table, idx3d):
    mesh = plsc.VectorSubcoreMesh(core_axis_name="c", subcore_axis_name="s")

    @functools.partial(
        pl.kernel,
        mesh=mesh,
        out_type=jax.ShapeDtypeStruct((_E, _D), jnp.float32),
        scratch_types=[
            pltpu.VMEM((_NCH, _CH), jnp.int32),
            pltpu.VMEM((_CH, _D), jnp.float32),
            pltpu.SemaphoreType.DMA,
        ],
    )
    def gather_kernel(table_hbm, idx_hbm, out_hbm, idx_v, rows_v, sem):
        wid = lax.axis_index("s") * _NC + lax.axis_index("c")
        ebase = wid * _RPW
        pltpu.sync_copy(idx_hbm.at[wid], idx_v)

        def body(t, carry):
            pltpu.async_copy(table_hbm.at[idx_v.at[t]], rows_v, sem).wait()
            pltpu.sync_copy(rows_v, out_hbm.at[pl.ds(ebase + t * _CH, _CH)])
            return carry

        lax.fori_loop(0, _NCH, body, 0)

    return gather_kernel(table, idx3d)


def _sc_scatter(m, idx3d, zinit):
    mesh = plsc.VectorSubcoreMesh(core_axis_name="c", subcore_axis_name="s")

    @functools.partial(
        pl.kernel,
        mesh=mesh,
        out_type=jax.ShapeDtypeStruct((_NC * _N, _D), jnp.float32),
        scratch_types=[
            pltpu.VMEM((_NCH, _CH), jnp.int32),
            pltpu.VMEM((_CH, _D), jnp.float32),
            pltpu.VMEM_SHARED((_N, _D), jnp.float32),
            pltpu.SemaphoreType.DMA,
        ],
    )
    def scatter_kernel(m_hbm, idx_hbm, z_hbm, out_hbm, idx_v, rows_v, acc, sem):
        c = lax.axis_index("c")
        s = lax.axis_index("s")
        wid = s * _NC + c
        ebase = wid * _RPW
        pltpu.sync_copy(idx_hbm.at[wid], idx_v)
        pltpu.sync_copy(z_hbm.at[pl.ds(0, _TPS)], acc.at[pl.ds(s * _TPS, _TPS)])

        @pl.when(s == _NS - 1)
        def _():
            pltpu.sync_copy(
                z_hbm.at[pl.ds(0, _TAIL)], acc.at[pl.ds(_NS * _TPS, _TAIL)]
            )

        plsc.subcore_barrier()

        def body(t, carry):
            pltpu.async_copy(
                m_hbm.at[pl.ds(ebase + t * _CH, _CH)], rows_v, sem
            ).wait()
            pltpu.sync_copy(rows_v, acc.at[idx_v.at[t]], add=True)
            return carry

        lax.fori_loop(0, _NCH, body, 0)
        plsc.subcore_barrier()
        pltpu.sync_copy(
            acc.at[pl.ds(s * _TPS, _TPS)],
            out_hbm.at[pl.ds(c * _N + s * _TPS, _TPS)],
        )

        @pl.when(s == _NS - 1)
        def _():
            pltpu.sync_copy(
                acc.at[pl.ds(_NS * _TPS, _TAIL)],
                out_hbm.at[pl.ds(c * _N + _NS * _TPS, _TAIL)],
            )

    return scatter_kernel(m, idx3d, zinit)


def kernel(x, edge_index, edge_features, Wn, bn, We, be, Wm, bm, Wlin, blin):
    src3d = edge_index[0].astype(jnp.int32).reshape(_NW, _NCH, _CH)
    dst3d = edge_index[1].astype(jnp.int32).reshape(_NW, _NCH, _CH)
    zinit = jnp.zeros((_TPS + _TAIL, _D), jnp.float32)

    hn = _tc_node0(x, Wn[0], bn[0].reshape(1, _D))
    parts = None
    for i in range(_L):
        if i > 0:
            hn = _tc_node_upd(
                hn, parts[:_N], parts[_N:], Wn[i], bn[i].reshape(1, _D)
            )
        g = _sc_gather(hn, src3d)
        m = _tc_edge(
            g, edge_features, We[i], be[i].reshape(1, _D), Wm[i],
            bm[i].reshape(1, _D),
        )
        parts = _sc_scatter(m, dst3d, zinit)

    out = _tc_final(hn, parts[:_N], parts[_N:], Wlin, blin.reshape(1, _EMB))
    return out.reshape(_EMB)


# speedup vs baseline: 3.0579x; 1.2409x over previous
import functools

import jax
import jax.numpy as jnp
from jax import lax
from jax.experimental import pallas as pl
from jax.experimental.pallas import tpu as pltpu
from jax.experimental.pallas import tpu_sc as plsc

_N = 10000
_E = 320000
_D = 128
_DE = 16
_L = 4
_EMB = 128

_NC = 2
_NS = 16
_NW = _NC * _NS
_CH = 80
_RPW = _E // _NW
_NCH = _RPW // _CH
_TPS = 624
_TAIL = _N - _NS * _TPS

_RB = 2000
_BE = 2000

_NBUF = 5
_NGRP = _NCH // _NBUF

_SCH = 40
_SNCH = _RPW // _SCH
_SNBUF = 2
_SNGRP = _SNCH // _SNBUF


def _node0_body(h_ref, w_ref, b_ref, o_ref):
    o_ref[...] = (
        jnp.dot(h_ref[...], w_ref[...], preferred_element_type=jnp.float32)
        + b_ref[...]
    )


def _tc_node0(h, w, b):
    return pl.pallas_call(
        _node0_body,
        grid=(_N // _RB,),
        in_specs=[
            pl.BlockSpec((_RB, _D), lambda i: (i, 0)),
            pl.BlockSpec((_D, _D), lambda i: (0, 0)),
            pl.BlockSpec((1, _D), lambda i: (0, 0)),
        ],
        out_specs=pl.BlockSpec((_RB, _D), lambda i: (i, 0)),
        out_shape=jax.ShapeDtypeStruct((_N, _D), jnp.float32),
    )(h, w, b)


def _node_upd_body(hn_ref, pa_ref, pb_ref, w_ref, b_ref, o_ref):
    h = hn_ref[...] + pa_ref[...] + pb_ref[...]
    o_ref[...] = (
        jnp.dot(h, w_ref[...], preferred_element_type=jnp.float32) + b_ref[...]
    )


def _tc_node_upd(hn, pa, pb, w, b):
    return pl.pallas_call(
        _node_upd_body,
        grid=(_N // _RB,),
        in_specs=[
            pl.BlockSpec((_RB, _D), lambda i: (i, 0)),
            pl.BlockSpec((_RB, _D), lambda i: (i, 0)),
            pl.BlockSpec((_RB, _D), lambda i: (i, 0)),
            pl.BlockSpec((_D, _D), lambda i: (0, 0)),
            pl.BlockSpec((1, _D), lambda i: (0, 0)),
        ],
        out_specs=pl.BlockSpec((_RB, _D), lambda i: (i, 0)),
        out_shape=jax.ShapeDtypeStruct((_N, _D), jnp.float32),
    )(hn, pa, pb, w, b)


def _edge_body(g_ref, ef_ref, we_ref, be_ref, wm_ref, bm_ref, o_ref):
    me = (
        jnp.dot(ef_ref[...], we_ref[...], preferred_element_type=jnp.float32)
        + be_ref[...]
    )
    t = g_ref[...] * me
    o_ref[...] = jnp.tanh(
        jnp.dot(t, wm_ref[...], preferred_element_type=jnp.float32) + bm_ref[...]
    )


def _tc_edge(g, ef, we, be, wm, bm):
    return pl.pallas_call(
        _edge_body,
        grid=(_E // _BE,),
        in_specs=[
            pl.BlockSpec((_BE, _D), lambda i: (i, 0)),
            pl.BlockSpec((_BE, _DE), lambda i: (i, 0)),
            pl.BlockSpec((_DE, _D), lambda i: (0, 0)),
            pl.BlockSpec((1, _D), lambda i: (0, 0)),
            pl.BlockSpec((_D, _D), lambda i: (0, 0)),
            pl.BlockSpec((1, _D), lambda i: (0, 0)),
        ],
        out_specs=pl.BlockSpec((_BE, _D), lambda i: (i, 0)),
        out_shape=jax.ShapeDtypeStruct((_E, _D), jnp.float32),
    )(g, ef, we, be, wm, bm)


def _final_body(hn_ref, pa_ref, pb_ref, wl_ref, bl_ref, o_ref):
    h = hn_ref[...] + pa_ref[...] + pb_ref[...]
    s = jnp.sum(h, axis=0, keepdims=True) * (1.0 / _N)
    o_ref[...] = (
        jnp.dot(s, wl_ref[...], preferred_element_type=jnp.float32) + bl_ref[...]
    )


def _tc_final(hn, pa, pb, wl, bl):
    return pl.pallas_call(
        _final_body,
        out_shape=jax.ShapeDtypeStruct((1, _EMB), jnp.float32),
    )(hn, pa, pb, wl, bl)


def _sc_gather(table, idx3d):
    mesh = plsc.VectorSubcoreMesh(core_axis_name="c", subcore_axis_name="s")

    @functools.partial(
        pl.kernel,
        mesh=mesh,
        out_type=jax.ShapeDtypeStruct((_E, _D), jnp.float32),
        scratch_types=[
            pltpu.VMEM((_NCH, _CH), jnp.int32),
            pltpu.VMEM((_NBUF, _CH, _D), jnp.float32),
            pltpu.SemaphoreType.DMA,
            pltpu.SemaphoreType.DMA,
        ],
    )
    def gather_kernel(table_hbm, idx_hbm, out_hbm, idx_v, rows_v, gsem, osem):
        wid = lax.axis_index("s") * _NC + lax.axis_index("c")
        ebase = wid * _RPW
        pltpu.sync_copy(idx_hbm.at[wid], idx_v)
        for b in range(_NBUF):
            pltpu.async_copy(table_hbm.at[idx_v.at[b]], rows_v.at[b], gsem)

        def outer(grp, carry):
            for b in range(_NBUF):
                t = grp * _NBUF + b
                pltpu.make_async_copy(
                    table_hbm.at[idx_v.at[t]], rows_v.at[b], gsem
                ).wait()
                pltpu.async_copy(
                    rows_v.at[b], out_hbm.at[pl.ds(ebase + t * _CH, _CH)], osem
                )
            for b in range(_NBUF):
                t = grp * _NBUF + b
                pltpu.make_async_copy(
                    rows_v.at[b], out_hbm.at[pl.ds(ebase + t * _CH, _CH)], osem
                ).wait()

                @pl.when(grp + 1 < _NGRP)
                def _():
                    t2 = (grp + 1) * _NBUF + b
                    pltpu.async_copy(
                        table_hbm.at[idx_v.at[t2]], rows_v.at[b], gsem
                    )
            return carry

        lax.fori_loop(0, _NGRP, outer, 0)

    return gather_kernel(table, idx3d)


def _sc_scatter(m, idx3d, zinit):
    mesh = plsc.VectorSubcoreMesh(core_axis_name="c", subcore_axis_name="s")

    @functools.partial(
        pl.kernel,
        mesh=mesh,
        out_type=jax.ShapeDtypeStruct((_NC * _N, _D), jnp.float32),
        scratch_types=[
            pltpu.VMEM((_SNCH, _SCH), jnp.int32),
            pltpu.VMEM((_SNBUF, _SCH, _D), jnp.float32),
            pltpu.VMEM_SHARED((_N, _D), jnp.float32),
            pltpu.SemaphoreType.DMA,
        ],
    )
    def scatter_kernel(m_hbm, idx_hbm, z_hbm, out_hbm, idx_v, rows_v, acc, sem):
        c = lax.axis_index("c")
        s = lax.axis_index("s")
        wid = s * _NC + c
        ebase = wid * _RPW
        pltpu.sync_copy(idx_hbm.at[wid], idx_v)
        pltpu.sync_copy(z_hbm.at[pl.ds(0, _TPS)], acc.at[pl.ds(s * _TPS, _TPS)])

        @pl.when(s == _NS - 1)
        def _():
            pltpu.sync_copy(
                z_hbm.at[pl.ds(0, _TAIL)], acc.at[pl.ds(_NS * _TPS, _TAIL)]
            )

        plsc.subcore_barrier()
        for b in range(_SNBUF):
            pltpu.async_copy(
                m_hbm.at[pl.ds(ebase + b * _SCH, _SCH)], rows_v.at[b], sem
            )

        def outer(grp, carry):
            for b in range(_SNBUF):
                t = grp * _SNBUF + b
                pltpu.make_async_copy(
                    m_hbm.at[pl.ds(ebase + t * _SCH, _SCH)], rows_v.at[b], sem
                ).wait()
                pltpu.sync_copy(rows_v.at[b], acc.at[idx_v.at[t]], add=True)

                @pl.when(grp + 1 < _SNGRP)
                def _():
                    t2 = (grp + 1) * _SNBUF + b
                    pltpu.async_copy(
                        m_hbm.at[pl.ds(ebase + t2 * _SCH, _SCH)], rows_v.at[b],
                        sem,
                    )
            return carry

        lax.fori_loop(0, _SNGRP, outer, 0)
        plsc.subcore_barrier()
        pltpu.sync_copy(
            acc.at[pl.ds(s * _TPS, _TPS)],
            out_hbm.at[pl.ds(c * _N + s * _TPS, _TPS)],
        )

        @pl.when(s == _NS - 1)
        def _():
            pltpu.sync_copy(
                acc.at[pl.ds(_NS * _TPS, _TAIL)],
                out_hbm.at[pl.ds(c * _N + _NS * _TPS, _TAIL)],
            )

    return scatter_kernel(m, idx3d, zinit)


def kernel(x, edge_index, edge_features, Wn, bn, We, be, Wm, bm, Wlin, blin):
    src3d = edge_index[0].astype(jnp.int32).reshape(_NW, _NCH, _CH)
    dst3d = edge_index[1].astype(jnp.int32).reshape(_NW, _SNCH, _SCH)
    zinit = jnp.zeros((_TPS + _TAIL, _D), jnp.float32)

    hn = _tc_node0(x, Wn[0], bn[0].reshape(1, _D))
    parts = None
    for i in range(_L):
        if i > 0:
            hn = _tc_node_upd(
                hn, parts[:_N], parts[_N:], Wn[i], bn[i].reshape(1, _D)
            )
        g = _sc_gather(hn, src3d)
        m = _tc_edge(
            g, edge_features, We[i], be[i].reshape(1, _D), Wm[i],
            bm[i].reshape(1, _D),
        )
        parts = _sc_scatter(m, dst3d, zinit)

    out = _tc_final(hn, parts[:_N], parts[_N:], Wlin, blin.reshape(1, _EMB))
    return out.reshape(_EMB)


# speedup vs baseline: 3.3235x; 1.0869x over previous
import functools

import jax
import jax.numpy as jnp
from jax import lax
from jax.experimental import pallas as pl
from jax.experimental.pallas import tpu as pltpu
from jax.experimental.pallas import tpu_sc as plsc

_N = 10000
_E = 320000
_D = 128
_DE = 16
_L = 4
_EMB = 128

_NC = 2
_NS = 16
_NW = _NC * _NS
_CH = 40
_RPW = _E // _NW
_NCH = _RPW // _CH
_GNBUF = 2
_GNGRP = _NCH // _GNBUF
_TPS = 624
_TAIL = _N - _NS * _TPS

_RB = 2000
_BE = 2000

_NBUF = 5
_NGRP = _NCH // _NBUF

_SCH = 40
_SNCH = _RPW // _SCH
_SNBUF = 2
_SNGRP = _SNCH // _SNBUF


def _node0_body(h_ref, w_ref, b_ref, o_ref):
    o_ref[...] = (
        jnp.dot(h_ref[...], w_ref[...], preferred_element_type=jnp.float32)
        + b_ref[...]
    )


def _tc_node0(h, w, b):
    return pl.pallas_call(
        _node0_body,
        grid=(_N // _RB,),
        in_specs=[
            pl.BlockSpec((_RB, _D), lambda i: (i, 0)),
            pl.BlockSpec((_D, _D), lambda i: (0, 0)),
            pl.BlockSpec((1, _D), lambda i: (0, 0)),
        ],
        out_specs=pl.BlockSpec((_RB, _D), lambda i: (i, 0)),
        out_shape=jax.ShapeDtypeStruct((_N, _D), jnp.float32),
    )(h, w, b)


def _node_upd_body(hn_ref, pa_ref, pb_ref, w_ref, b_ref, o_ref):
    h = hn_ref[...] + pa_ref[...] + pb_ref[...]
    o_ref[...] = (
        jnp.dot(h, w_ref[...], preferred_element_type=jnp.float32) + b_ref[...]
    )


def _tc_node_upd(hn, pa, pb, w, b):
    return pl.pallas_call(
        _node_upd_body,
        grid=(_N // _RB,),
        in_specs=[
            pl.BlockSpec((_RB, _D), lambda i: (i, 0)),
            pl.BlockSpec((_RB, _D), lambda i: (i, 0)),
            pl.BlockSpec((_RB, _D), lambda i: (i, 0)),
            pl.BlockSpec((_D, _D), lambda i: (0, 0)),
            pl.BlockSpec((1, _D), lambda i: (0, 0)),
        ],
        out_specs=pl.BlockSpec((_RB, _D), lambda i: (i, 0)),
        out_shape=jax.ShapeDtypeStruct((_N, _D), jnp.float32),
    )(hn, pa, pb, w, b)


def _edge_body(g_ref, ef_ref, we_ref, be_ref, wm_ref, bm_ref, o_ref):
    me = (
        jnp.dot(ef_ref[...], we_ref[...], preferred_element_type=jnp.float32)
        + be_ref[...]
    )
    t = g_ref[...] * me
    o_ref[...] = jnp.tanh(
        jnp.dot(t, wm_ref[...], preferred_element_type=jnp.float32) + bm_ref[...]
    )


def _tc_edge(g, ef, we, be, wm, bm):
    return pl.pallas_call(
        _edge_body,
        grid=(_E // _BE,),
        in_specs=[
            pl.BlockSpec((_BE, _D), lambda i: (i, 0)),
            pl.BlockSpec((_BE, _DE), lambda i: (i, 0)),
            pl.BlockSpec((_DE, _D), lambda i: (0, 0)),
            pl.BlockSpec((1, _D), lambda i: (0, 0)),
            pl.BlockSpec((_D, _D), lambda i: (0, 0)),
            pl.BlockSpec((1, _D), lambda i: (0, 0)),
        ],
        out_specs=pl.BlockSpec((_BE, _D), lambda i: (i, 0)),
        out_shape=jax.ShapeDtypeStruct((_E, _D), jnp.float32),
    )(g, ef, we, be, wm, bm)


def _final_body(hn_ref, pa_ref, pb_ref, wl_ref, bl_ref, o_ref):
    h = hn_ref[...] + pa_ref[...] + pb_ref[...]
    s = jnp.sum(h, axis=0, keepdims=True) * (1.0 / _N)
    o_ref[...] = (
        jnp.dot(s, wl_ref[...], preferred_element_type=jnp.float32) + bl_ref[...]
    )


def _tc_final(hn, pa, pb, wl, bl):
    return pl.pallas_call(
        _final_body,
        out_shape=jax.ShapeDtypeStruct((1, _EMB), jnp.float32),
    )(hn, pa, pb, wl, bl)


def _sc_gather(table, idx3d):
    mesh = plsc.VectorSubcoreMesh(core_axis_name="c", subcore_axis_name="s")

    @functools.partial(
        pl.kernel,
        mesh=mesh,
        out_type=jax.ShapeDtypeStruct((_E, _D), jnp.float32),
        scratch_types=[
            pltpu.VMEM((_NCH, _CH), jnp.int32),
            pltpu.VMEM((_GNBUF, _CH, _D), jnp.float32),
            pltpu.VMEM_SHARED((_N, _D), jnp.float32),
            pltpu.SemaphoreType.DMA,
            pltpu.SemaphoreType.DMA,
        ],
    )
    def gather_kernel(table_hbm, idx_hbm, out_hbm, idx_v, rows_v, tab, gsem,
                      osem):
        s = lax.axis_index("s")
        wid = s * _NC + lax.axis_index("c")
        ebase = wid * _RPW
        pltpu.sync_copy(idx_hbm.at[wid], idx_v)
        pltpu.sync_copy(
            table_hbm.at[pl.ds(s * _TPS, _TPS)], tab.at[pl.ds(s * _TPS, _TPS)]
        )

        @pl.when(s == _NS - 1)
        def _():
            pltpu.sync_copy(
                table_hbm.at[pl.ds(_NS * _TPS, _TAIL)],
                tab.at[pl.ds(_NS * _TPS, _TAIL)],
            )

        plsc.subcore_barrier()
        for b in range(_GNBUF):
            pltpu.async_copy(tab.at[idx_v.at[b]], rows_v.at[b], gsem)

        def outer(grp, carry):
            for b in range(_GNBUF):
                t = grp * _GNBUF + b
                pltpu.make_async_copy(
                    tab.at[idx_v.at[t]], rows_v.at[b], gsem
                ).wait()
                pltpu.async_copy(
                    rows_v.at[b], out_hbm.at[pl.ds(ebase + t * _CH, _CH)], osem
                )
            for b in range(_GNBUF):
                t = grp * _GNBUF + b
                pltpu.make_async_copy(
                    rows_v.at[b], out_hbm.at[pl.ds(ebase + t * _CH, _CH)], osem
                ).wait()

                @pl.when(grp + 1 < _GNGRP)
                def _():
                    t2 = (grp + 1) * _GNBUF + b
                    pltpu.async_copy(tab.at[idx_v.at[t2]], rows_v.at[b], gsem)
            return carry

        lax.fori_loop(0, _GNGRP, outer, 0)

    return gather_kernel(table, idx3d)


def _sc_scatter(m, idx3d, zinit):
    mesh = plsc.VectorSubcoreMesh(core_axis_name="c", subcore_axis_name="s")

    @functools.partial(
        pl.kernel,
        mesh=mesh,
        out_type=jax.ShapeDtypeStruct((_NC * _N, _D), jnp.float32),
        scratch_types=[
            pltpu.VMEM((_SNCH, _SCH), jnp.int32),
            pltpu.VMEM((_SNBUF, _SCH, _D), jnp.float32),
            pltpu.VMEM_SHARED((_N, _D), jnp.float32),
            pltpu.SemaphoreType.DMA,
        ],
    )
    def scatter_kernel(m_hbm, idx_hbm, z_hbm, out_hbm, idx_v, rows_v, acc, sem):
        c = lax.axis_index("c")
        s = lax.axis_index("s")
        wid = s * _NC + c
        ebase = wid * _RPW
        pltpu.sync_copy(idx_hbm.at[wid], idx_v)
        pltpu.sync_copy(z_hbm.at[pl.ds(0, _TPS)], acc.at[pl.ds(s * _TPS, _TPS)])

        @pl.when(s == _NS - 1)
        def _():
            pltpu.sync_copy(
                z_hbm.at[pl.ds(0, _TAIL)], acc.at[pl.ds(_NS * _TPS, _TAIL)]
            )

        plsc.subcore_barrier()
        for b in range(_SNBUF):
            pltpu.async_copy(
                m_hbm.at[pl.ds(ebase + b * _SCH, _SCH)], rows_v.at[b], sem
            )

        def outer(grp, carry):
            for b in range(_SNBUF):
                t = grp * _SNBUF + b
                pltpu.make_async_copy(
                    m_hbm.at[pl.ds(ebase + t * _SCH, _SCH)], rows_v.at[b], sem
                ).wait()
                pltpu.sync_copy(rows_v.at[b], acc.at[idx_v.at[t]], add=True)

                @pl.when(grp + 1 < _SNGRP)
                def _():
                    t2 = (grp + 1) * _SNBUF + b
                    pltpu.async_copy(
                        m_hbm.at[pl.ds(ebase + t2 * _SCH, _SCH)], rows_v.at[b],
                        sem,
                    )
            return carry

        lax.fori_loop(0, _SNGRP, outer, 0)
        plsc.subcore_barrier()
        pltpu.sync_copy(
            acc.at[pl.ds(s * _TPS, _TPS)],
            out_hbm.at[pl.ds(c * _N + s * _TPS, _TPS)],
        )

        @pl.when(s == _NS - 1)
        def _():
            pltpu.sync_copy(
                acc.at[pl.ds(_NS * _TPS, _TAIL)],
                out_hbm.at[pl.ds(c * _N + _NS * _TPS, _TAIL)],
            )

    return scatter_kernel(m, idx3d, zinit)


def kernel(x, edge_index, edge_features, Wn, bn, We, be, Wm, bm, Wlin, blin):
    src3d = edge_index[0].astype(jnp.int32).reshape(_NW, _NCH, _CH)
    dst3d = edge_index[1].astype(jnp.int32).reshape(_NW, _SNCH, _SCH)
    zinit = jnp.zeros((_TPS + _TAIL, _D), jnp.float32)

    hn = _tc_node0(x, Wn[0], bn[0].reshape(1, _D))
    parts = None
    for i in range(_L):
        if i > 0:
            hn = _tc_node_upd(
                hn, parts[:_N], parts[_N:], Wn[i], bn[i].reshape(1, _D)
            )
        g = _sc_gather(hn, src3d)
        m = _tc_edge(
            g, edge_features, We[i], be[i].reshape(1, _D), Wm[i],
            bm[i].reshape(1, _D),
        )
        parts = _sc_scatter(m, dst3d, zinit)

    out = _tc_final(hn, parts[:_N], parts[_N:], Wlin, blin.reshape(1, _EMB))
    return out.reshape(_EMB)


# speedup vs baseline: 3.6321x; 1.0928x over previous
import functools

import jax
import jax.numpy as jnp
from jax import lax
from jax.experimental import pallas as pl
from jax.experimental.pallas import tpu as pltpu
from jax.experimental.pallas import tpu_sc as plsc

_N = 10000
_E = 320000
_D = 128
_DE = 16
_L = 4
_EMB = 128

_NC = 2
_NS = 16
_NW = _NC * _NS
_CH = 40
_RPW = _E // _NW
_NCH = _RPW // _CH
_GNBUF = 2
_GNGRP = _NCH // _GNBUF
_TPS = 624
_TAIL = _N - _NS * _TPS

_RB = 2000
_BE = 2000

_NBUF = 5
_NGRP = _NCH // _NBUF

_SCH = 40
_SNCH = _RPW // _SCH
_SNBUF = 2
_SNGRP = _SNCH // _SNBUF


def _node0_body(h_ref, w_ref, b_ref, o_ref):
    o_ref[...] = (
        jnp.dot(h_ref[...], w_ref[...], preferred_element_type=jnp.float32)
        + b_ref[...]
    )


def _tc_node0(h, w, b):
    return pl.pallas_call(
        _node0_body,
        grid=(_N // _RB,),
        in_specs=[
            pl.BlockSpec((_RB, _D), lambda i: (i, 0)),
            pl.BlockSpec((_D, _D), lambda i: (0, 0)),
            pl.BlockSpec((1, _D), lambda i: (0, 0)),
        ],
        out_specs=pl.BlockSpec((_RB, _D), lambda i: (i, 0)),
        out_shape=jax.ShapeDtypeStruct((_N, _D), jnp.float32),
    )(h, w, b)


def _node_upd_body(hn_ref, p1_ref, p2_ref, p3_ref, p4_ref, w_ref, b_ref, o_ref):
    h = (
        hn_ref[...]
        + (p1_ref[...] + p2_ref[...])
        + (p3_ref[...] + p4_ref[...])
    )
    o_ref[...] = (
        jnp.dot(h, w_ref[...], preferred_element_type=jnp.float32) + b_ref[...]
    )


def _tc_node_upd(hn, p1, p2, p3, p4, w, b):
    return pl.pallas_call(
        _node_upd_body,
        grid=(_N // _RB,),
        in_specs=[
            pl.BlockSpec((_RB, _D), lambda i: (i, 0)),
            pl.BlockSpec((_RB, _D), lambda i: (i, 0)),
            pl.BlockSpec((_RB, _D), lambda i: (i, 0)),
            pl.BlockSpec((_RB, _D), lambda i: (i, 0)),
            pl.BlockSpec((_RB, _D), lambda i: (i, 0)),
            pl.BlockSpec((_D, _D), lambda i: (0, 0)),
            pl.BlockSpec((1, _D), lambda i: (0, 0)),
        ],
        out_specs=pl.BlockSpec((_RB, _D), lambda i: (i, 0)),
        out_shape=jax.ShapeDtypeStruct((_N, _D), jnp.float32),
    )(hn, p1, p2, p3, p4, w, b)


def _edge_body(g_ref, ef_ref, we_ref, be_ref, wm_ref, bm_ref, o_ref):
    me = (
        jnp.dot(ef_ref[...], we_ref[...], preferred_element_type=jnp.float32)
        + be_ref[...]
    )
    t = g_ref[...] * me
    o_ref[...] = jnp.tanh(
        jnp.dot(t, wm_ref[...], preferred_element_type=jnp.float32) + bm_ref[...]
    )


def _tc_edge(g, ef, we, be, wm, bm, ne):
    return pl.pallas_call(
        _edge_body,
        grid=(ne // _BE,),
        in_specs=[
            pl.BlockSpec((_BE, _D), lambda i: (i, 0)),
            pl.BlockSpec((_BE, _DE), lambda i: (i, 0)),
            pl.BlockSpec((_DE, _D), lambda i: (0, 0)),
            pl.BlockSpec((1, _D), lambda i: (0, 0)),
            pl.BlockSpec((_D, _D), lambda i: (0, 0)),
            pl.BlockSpec((1, _D), lambda i: (0, 0)),
        ],
        out_specs=pl.BlockSpec((_BE, _D), lambda i: (i, 0)),
        out_shape=jax.ShapeDtypeStruct((ne, _D), jnp.float32),
    )(g, ef, we, be, wm, bm)


def _final_body(hn_ref, p1_ref, p2_ref, p3_ref, p4_ref, wl_ref, bl_ref, o_ref):
    h = (
        hn_ref[...]
        + (p1_ref[...] + p2_ref[...])
        + (p3_ref[...] + p4_ref[...])
    )
    s = jnp.sum(h, axis=0, keepdims=True) * (1.0 / _N)
    o_ref[...] = (
        jnp.dot(s, wl_ref[...], preferred_element_type=jnp.float32) + bl_ref[...]
    )


def _tc_final(hn, p1, p2, p3, p4, wl, bl):
    return pl.pallas_call(
        _final_body,
        out_shape=jax.ShapeDtypeStruct((1, _EMB), jnp.float32),
    )(hn, p1, p2, p3, p4, wl, bl)


def _sc_gather(table, idx3d, ne):
    rpw = ne // _NW
    nch = rpw // _CH
    ngrp = nch // _GNBUF
    mesh = plsc.VectorSubcoreMesh(core_axis_name="c", subcore_axis_name="s")

    @functools.partial(
        pl.kernel,
        mesh=mesh,
        out_type=jax.ShapeDtypeStruct((ne, _D), jnp.float32),
        scratch_types=[
            pltpu.VMEM((nch, _CH), jnp.int32),
            pltpu.VMEM((_GNBUF, _CH, _D), jnp.float32),
            pltpu.VMEM_SHARED((_N, _D), jnp.float32),
            pltpu.SemaphoreType.DMA,
            pltpu.SemaphoreType.DMA,
        ],
    )
    def gather_kernel(table_hbm, idx_hbm, out_hbm, idx_v, rows_v, tab, gsem,
                      osem):
        s = lax.axis_index("s")
        wid = s * _NC + lax.axis_index("c")
        ebase = wid * rpw
        pltpu.sync_copy(idx_hbm.at[wid], idx_v)
        pltpu.sync_copy(
            table_hbm.at[pl.ds(s * _TPS, _TPS)], tab.at[pl.ds(s * _TPS, _TPS)]
        )

        @pl.when(s == _NS - 1)
        def _():
            pltpu.sync_copy(
                table_hbm.at[pl.ds(_NS * _TPS, _TAIL)],
                tab.at[pl.ds(_NS * _TPS, _TAIL)],
            )

        plsc.subcore_barrier()
        for b in range(_GNBUF):
            pltpu.async_copy(tab.at[idx_v.at[b]], rows_v.at[b], gsem)

        def outer(grp, carry):
            for b in range(_GNBUF):
                t = grp * _GNBUF + b
                pltpu.make_async_copy(
                    tab.at[idx_v.at[t]], rows_v.at[b], gsem
                ).wait()
                pltpu.async_copy(
                    rows_v.at[b], out_hbm.at[pl.ds(ebase + t * _CH, _CH)], osem
                )
            for b in range(_GNBUF):
                t = grp * _GNBUF + b
                pltpu.make_async_copy(
                    rows_v.at[b], out_hbm.at[pl.ds(ebase + t * _CH, _CH)], osem
                ).wait()

                @pl.when(grp + 1 < ngrp)
                def _():
                    t2 = (grp + 1) * _GNBUF + b
                    pltpu.async_copy(tab.at[idx_v.at[t2]], rows_v.at[b], gsem)
            return carry

        lax.fori_loop(0, ngrp, outer, 0)

    return gather_kernel(table, idx3d)


def _sc_scatter(m, idx3d, zinit, ne):
    rpw = ne // _NW
    snch = rpw // _SCH
    sngrp = snch // _SNBUF
    mesh = plsc.VectorSubcoreMesh(core_axis_name="c", subcore_axis_name="s")

    @functools.partial(
        pl.kernel,
        mesh=mesh,
        out_type=jax.ShapeDtypeStruct((_NC * _N, _D), jnp.float32),
        scratch_types=[
            pltpu.VMEM((snch, _SCH), jnp.int32),
            pltpu.VMEM((_SNBUF, _SCH, _D), jnp.float32),
            pltpu.VMEM_SHARED((_N, _D), jnp.float32),
            pltpu.SemaphoreType.DMA,
        ],
    )
    def scatter_kernel(m_hbm, idx_hbm, z_hbm, out_hbm, idx_v, rows_v, acc, sem):
        c = lax.axis_index("c")
        s = lax.axis_index("s")
        wid = s * _NC + c
        ebase = wid * rpw
        pltpu.sync_copy(idx_hbm.at[wid], idx_v)
        pltpu.sync_copy(z_hbm.at[pl.ds(0, _TPS)], acc.at[pl.ds(s * _TPS, _TPS)])

        @pl.when(s == _NS - 1)
        def _():
            pltpu.sync_copy(
                z_hbm.at[pl.ds(0, _TAIL)], acc.at[pl.ds(_NS * _TPS, _TAIL)]
            )

        plsc.subcore_barrier()
        for b in range(_SNBUF):
            pltpu.async_copy(
                m_hbm.at[pl.ds(ebase + b * _SCH, _SCH)], rows_v.at[b], sem
            )

        def outer(grp, carry):
            for b in range(_SNBUF):
                t = grp * _SNBUF + b
                pltpu.make_async_copy(
                    m_hbm.at[pl.ds(ebase + t * _SCH, _SCH)], rows_v.at[b], sem
                ).wait()
                pltpu.sync_copy(rows_v.at[b], acc.at[idx_v.at[t]], add=True)

                @pl.when(grp + 1 < sngrp)
                def _():
                    t2 = (grp + 1) * _SNBUF + b
                    pltpu.async_copy(
                        m_hbm.at[pl.ds(ebase + t2 * _SCH, _SCH)], rows_v.at[b],
                        sem,
                    )
            return carry

        lax.fori_loop(0, sngrp, outer, 0)
        plsc.subcore_barrier()
        pltpu.sync_copy(
            acc.at[pl.ds(s * _TPS, _TPS)],
            out_hbm.at[pl.ds(c * _N + s * _TPS, _TPS)],
        )

        @pl.when(s == _NS - 1)
        def _():
            pltpu.sync_copy(
                acc.at[pl.ds(_NS * _TPS, _TAIL)],
                out_hbm.at[pl.ds(c * _N + _NS * _TPS, _TAIL)],
            )

    return scatter_kernel(m, idx3d, zinit)


_EA = 192000
_EB = _E - _EA


def kernel(x, edge_index, edge_features, Wn, bn, We, be, Wm, bm, Wlin, blin):
    src = edge_index[0].astype(jnp.int32)
    dst = edge_index[1].astype(jnp.int32)
    srcA = src[:_EA].reshape(_NW, _EA // _NW // _CH, _CH)
    srcB = src[_EA:].reshape(_NW, _EB // _NW // _CH, _CH)
    dstA = dst[:_EA].reshape(_NW, _EA // _NW // _SCH, _SCH)
    dstB = dst[_EA:].reshape(_NW, _EB // _NW // _SCH, _SCH)
    efA = edge_features[:_EA]
    efB = edge_features[_EA:]
    zinit = jnp.zeros((_TPS + _TAIL, _D), jnp.float32)

    hn = _tc_node0(x, Wn[0], bn[0].reshape(1, _D))
    pA = pB = None
    for i in range(_L):
        if i > 0:
            hn = _tc_node_upd(
                hn, pA[:_N], pA[_N:], pB[:_N], pB[_N:], Wn[i],
                bn[i].reshape(1, _D),
            )
        be_i = be[i].reshape(1, _D)
        bm_i = bm[i].reshape(1, _D)
        gA = _sc_gather(hn, srcA, _EA)
        gB = _sc_gather(hn, srcB, _EB)
        mA = _tc_edge(gA, efA, We[i], be_i, Wm[i], bm_i, _EA)
        mB = _tc_edge(gB, efB, We[i], be_i, Wm[i], bm_i, _EB)
        pA = _sc_scatter(mA, dstA, zinit, _EA)
        pB = _sc_scatter(mB, dstB, zinit, _EB)

    out = _tc_final(
        hn, pA[:_N], pA[_N:], pB[:_N], pB[_N:], Wlin, blin.reshape(1, _EMB)
    )
    return out.reshape(_EMB)


# speedup vs baseline: 3.9526x; 1.0882x over previous
import functools

import jax
import jax.numpy as jnp
from jax import lax
from jax.experimental import pallas as pl
from jax.experimental.pallas import tpu as pltpu
from jax.experimental.pallas import tpu_sc as plsc

_N = 10000
_E = 320000
_D = 128
_DE = 16
_L = 4
_EMB = 128

_NC = 2
_NS = 16
_NW = _NC * _NS
_CH = 80
_RPW = _E // _NW
_GNBUF = 2
_TPS = 624
_TAIL = _N - _NS * _TPS

_RB = 2000
_BE = 2000


_SCH = 80
_SNBUF = 2


def _node0_body(h_ref, w_ref, b_ref, o_ref):
    o_ref[...] = (
        jnp.dot(h_ref[...], w_ref[...], preferred_element_type=jnp.float32)
        + b_ref[...]
    )


def _tc_node0(h, w, b):
    return pl.pallas_call(
        _node0_body,
        grid=(_N // _RB,),
        in_specs=[
            pl.BlockSpec((_RB, _D), lambda i: (i, 0)),
            pl.BlockSpec((_D, _D), lambda i: (0, 0)),
            pl.BlockSpec((1, _D), lambda i: (0, 0)),
        ],
        out_specs=pl.BlockSpec((_RB, _D), lambda i: (i, 0)),
        out_shape=jax.ShapeDtypeStruct((_N, _D), jnp.float32),
    )(h, w, b)


def _node_upd_body(hn_ref, p1_ref, p2_ref, p3_ref, p4_ref, w_ref, b_ref, o_ref):
    h = (
        hn_ref[...]
        + (p1_ref[...] + p2_ref[...])
        + (p3_ref[...] + p4_ref[...])
    )
    o_ref[...] = (
        jnp.dot(h, w_ref[...], preferred_element_type=jnp.float32) + b_ref[...]
    )


def _tc_node_upd(hn, p1, p2, p3, p4, w, b):
    return pl.pallas_call(
        _node_upd_body,
        grid=(_N // _RB,),
        in_specs=[
            pl.BlockSpec((_RB, _D), lambda i: (i, 0)),
            pl.BlockSpec((_RB, _D), lambda i: (i, 0)),
            pl.BlockSpec((_RB, _D), lambda i: (i, 0)),
            pl.BlockSpec((_RB, _D), lambda i: (i, 0)),
            pl.BlockSpec((_RB, _D), lambda i: (i, 0)),
            pl.BlockSpec((_D, _D), lambda i: (0, 0)),
            pl.BlockSpec((1, _D), lambda i: (0, 0)),
        ],
        out_specs=pl.BlockSpec((_RB, _D), lambda i: (i, 0)),
        out_shape=jax.ShapeDtypeStruct((_N, _D), jnp.float32),
    )(hn, p1, p2, p3, p4, w, b)


def _edge_body(g_ref, ef_ref, we_ref, be_ref, wm_ref, bm_ref, o_ref):
    me = (
        jnp.dot(ef_ref[...], we_ref[...], preferred_element_type=jnp.float32)
        + be_ref[...]
    )
    t = g_ref[...] * me
    o_ref[...] = jnp.tanh(
        jnp.dot(t, wm_ref[...], preferred_element_type=jnp.float32) + bm_ref[...]
    )


def _tc_edge(g, ef, we, be, wm, bm, ne):
    return pl.pallas_call(
        _edge_body,
        grid=(ne // _BE,),
        in_specs=[
            pl.BlockSpec((_BE, _D), lambda i: (i, 0)),
            pl.BlockSpec((_BE, _DE), lambda i: (i, 0)),
            pl.BlockSpec((_DE, _D), lambda i: (0, 0)),
            pl.BlockSpec((1, _D), lambda i: (0, 0)),
            pl.BlockSpec((_D, _D), lambda i: (0, 0)),
            pl.BlockSpec((1, _D), lambda i: (0, 0)),
        ],
        out_specs=pl.BlockSpec((_BE, _D), lambda i: (i, 0)),
        out_shape=jax.ShapeDtypeStruct((ne, _D), jnp.float32),
    )(g, ef, we, be, wm, bm)


def _final_body(hn_ref, p1_ref, p2_ref, p3_ref, p4_ref, wl_ref, bl_ref, o_ref):
    h = (
        hn_ref[...]
        + (p1_ref[...] + p2_ref[...])
        + (p3_ref[...] + p4_ref[...])
    )
    s = jnp.sum(h, axis=0, keepdims=True) * (1.0 / _N)
    o_ref[...] = (
        jnp.dot(s, wl_ref[...], preferred_element_type=jnp.float32) + bl_ref[...]
    )


def _tc_final(hn, p1, p2, p3, p4, wl, bl):
    return pl.pallas_call(
        _final_body,
        out_shape=jax.ShapeDtypeStruct((1, _EMB), jnp.float32),
    )(hn, p1, p2, p3, p4, wl, bl)


def _sc_gather(table, idx3d, ne):
    rpw = ne // _NW
    nch = rpw // _CH
    ngrp = nch // _GNBUF
    leftover = nch - ngrp * _GNBUF
    mesh = plsc.VectorSubcoreMesh(core_axis_name="c", subcore_axis_name="s")

    @functools.partial(
        pl.kernel,
        mesh=mesh,
        out_type=jax.ShapeDtypeStruct((ne, _D), jnp.float32),
        scratch_types=[
            pltpu.VMEM((nch, _CH), jnp.int32),
            pltpu.VMEM((_GNBUF, _CH, _D), jnp.float32),
            pltpu.VMEM_SHARED((_N, _D), jnp.float32),
            pltpu.SemaphoreType.DMA,
            pltpu.SemaphoreType.DMA,
        ],
    )
    def gather_kernel(table_hbm, idx_hbm, out_hbm, idx_v, rows_v, tab, gsem,
                      osem):
        s = lax.axis_index("s")
        wid = s * _NC + lax.axis_index("c")
        ebase = wid * rpw
        pltpu.sync_copy(idx_hbm.at[wid], idx_v)
        pltpu.sync_copy(
            table_hbm.at[pl.ds(s * _TPS, _TPS)], tab.at[pl.ds(s * _TPS, _TPS)]
        )

        @pl.when(s == _NS - 1)
        def _():
            pltpu.sync_copy(
                table_hbm.at[pl.ds(_NS * _TPS, _TAIL)],
                tab.at[pl.ds(_NS * _TPS, _TAIL)],
            )

        plsc.subcore_barrier()
        for b in range(_GNBUF):
            pltpu.async_copy(tab.at[idx_v.at[b]], rows_v.at[b], gsem)

        def outer(grp, carry):
            for b in range(_GNBUF):
                t = grp * _GNBUF + b
                pltpu.make_async_copy(
                    tab.at[idx_v.at[t]], rows_v.at[b], gsem
                ).wait()
                pltpu.async_copy(
                    rows_v.at[b], out_hbm.at[pl.ds(ebase + t * _CH, _CH)], osem
                )
            for b in range(_GNBUF):
                t = grp * _GNBUF + b
                pltpu.make_async_copy(
                    rows_v.at[b], out_hbm.at[pl.ds(ebase + t * _CH, _CH)], osem
                ).wait()

                @pl.when(grp + 1 < ngrp)
                def _():
                    t2 = (grp + 1) * _GNBUF + b
                    pltpu.async_copy(tab.at[idx_v.at[t2]], rows_v.at[b], gsem)
            return carry

        lax.fori_loop(0, ngrp, outer, 0)
        for t in range(ngrp * _GNBUF, ngrp * _GNBUF + leftover):
            pltpu.async_copy(tab.at[idx_v.at[t]], rows_v.at[0], gsem).wait()
            pltpu.sync_copy(rows_v.at[0], out_hbm.at[pl.ds(ebase + t * _CH, _CH)])

    return gather_kernel(table, idx3d)


def _sc_scatter(m, idx3d, zinit, ne):
    rpw = ne // _NW
    snch = rpw // _SCH
    sngrp = snch // _SNBUF
    sleft = snch - sngrp * _SNBUF
    mesh = plsc.VectorSubcoreMesh(core_axis_name="c", subcore_axis_name="s")

    @functools.partial(
        pl.kernel,
        mesh=mesh,
        out_type=jax.ShapeDtypeStruct((_NC * _N, _D), jnp.float32),
        scratch_types=[
            pltpu.VMEM((snch, _SCH), jnp.int32),
            pltpu.VMEM((_SNBUF, _SCH, _D), jnp.float32),
            pltpu.VMEM_SHARED((_N, _D), jnp.float32),
            pltpu.SemaphoreType.DMA,
        ],
    )
    def scatter_kernel(m_hbm, idx_hbm, z_hbm, out_hbm, idx_v, rows_v, acc, sem):
        c = lax.axis_index("c")
        s = lax.axis_index("s")
        wid = s * _NC + c
        ebase = wid * rpw
        pltpu.sync_copy(idx_hbm.at[wid], idx_v)
        pltpu.sync_copy(z_hbm.at[pl.ds(0, _TPS)], acc.at[pl.ds(s * _TPS, _TPS)])

        @pl.when(s == _NS - 1)
        def _():
            pltpu.sync_copy(
                z_hbm.at[pl.ds(0, _TAIL)], acc.at[pl.ds(_NS * _TPS, _TAIL)]
            )

        plsc.subcore_barrier()
        for b in range(_SNBUF):
            pltpu.async_copy(
                m_hbm.at[pl.ds(ebase + b * _SCH, _SCH)], rows_v.at[b], sem
            )

        def outer(grp, carry):
            for b in range(_SNBUF):
                t = grp * _SNBUF + b
                pltpu.make_async_copy(
                    m_hbm.at[pl.ds(ebase + t * _SCH, _SCH)], rows_v.at[b], sem
                ).wait()
                pltpu.sync_copy(rows_v.at[b], acc.at[idx_v.at[t]], add=True)

                @pl.when(grp + 1 < sngrp)
                def _():
                    t2 = (grp + 1) * _SNBUF + b
                    pltpu.async_copy(
                        m_hbm.at[pl.ds(ebase + t2 * _SCH, _SCH)], rows_v.at[b],
                        sem,
                    )
            return carry

        lax.fori_loop(0, sngrp, outer, 0)
        for t in range(sngrp * _SNBUF, sngrp * _SNBUF + sleft):
            pltpu.async_copy(
                m_hbm.at[pl.ds(ebase + t * _SCH, _SCH)], rows_v.at[0], sem
            ).wait()
            pltpu.sync_copy(rows_v.at[0], acc.at[idx_v.at[t]], add=True)
        plsc.subcore_barrier()
        pltpu.sync_copy(
            acc.at[pl.ds(s * _TPS, _TPS)],
            out_hbm.at[pl.ds(c * _N + s * _TPS, _TPS)],
        )

        @pl.when(s == _NS - 1)
        def _():
            pltpu.sync_copy(
                acc.at[pl.ds(_NS * _TPS, _TAIL)],
                out_hbm.at[pl.ds(c * _N + _NS * _TPS, _TAIL)],
            )

    return scatter_kernel(m, idx3d, zinit)


_EA = 192000
_EB = _E - _EA


def kernel(x, edge_index, edge_features, Wn, bn, We, be, Wm, bm, Wlin, blin):
    src = edge_index[0].astype(jnp.int32)
    dst = edge_index[1].astype(jnp.int32)
    srcA = src[:_EA].reshape(_NW, _EA // _NW // _CH, _CH)
    srcB = src[_EA:].reshape(_NW, _EB // _NW // _CH, _CH)
    dstA = dst[:_EA].reshape(_NW, _EA // _NW // _SCH, _SCH)
    dstB = dst[_EA:].reshape(_NW, _EB // _NW // _SCH, _SCH)
    efA = edge_features[:_EA]
    efB = edge_features[_EA:]
    zinit = jnp.zeros((_TPS + _TAIL, _D), jnp.float32)

    hn = _tc_node0(x, Wn[0], bn[0].reshape(1, _D))
    pA = pB = None
    for i in range(_L):
        if i > 0:
            hn = _tc_node_upd(
                hn, pA[:_N], pA[_N:], pB[:_N], pB[_N:], Wn[i],
                bn[i].reshape(1, _D),
            )
        be_i = be[i].reshape(1, _D)
        bm_i = bm[i].reshape(1, _D)
        gA = _sc_gather(hn, srcA, _EA)
        gB = _sc_gather(hn, srcB, _EB)
        mA = _tc_edge(gA, efA, We[i], be_i, Wm[i], bm_i, _EA)
        mB = _tc_edge(gB, efB, We[i], be_i, Wm[i], bm_i, _EB)
        pA = _sc_scatter(mA, dstA, zinit, _EA)
        pB = _sc_scatter(mB, dstB, zinit, _EB)

    out = _tc_final(
        hn, pA[:_N], pA[_N:], pB[:_N], pB[_N:], Wlin, blin.reshape(1, _EMB)
    )
    return out.reshape(_EMB)


# speedup vs baseline: 3.9999x; 1.0120x over previous
import functools

import jax
import jax.numpy as jnp
from jax import lax
from jax.experimental import pallas as pl
from jax.experimental.pallas import tpu as pltpu
from jax.experimental.pallas import tpu_sc as plsc

_N = 10000
_E = 320000
_D = 128
_DE = 16
_L = 4
_EMB = 128

_NC = 2
_NS = 16
_NW = _NC * _NS
_CH = 80
_RPW = _E // _NW
_GNBUF = 2
_TPS = 624
_TAIL = _N - _NS * _TPS

_RB = 2000
_BE = 2000


_SCH = 80
_SNBUF = 3


def _node0_body(h_ref, w_ref, b_ref, o_ref):
    o_ref[...] = (
        jnp.dot(h_ref[...], w_ref[...], preferred_element_type=jnp.float32)
        + b_ref[...]
    )


def _tc_node0(h, w, b):
    return pl.pallas_call(
        _node0_body,
        grid=(_N // _RB,),
        in_specs=[
            pl.BlockSpec((_RB, _D), lambda i: (i, 0)),
            pl.BlockSpec((_D, _D), lambda i: (0, 0)),
            pl.BlockSpec((1, _D), lambda i: (0, 0)),
        ],
        out_specs=pl.BlockSpec((_RB, _D), lambda i: (i, 0)),
        out_shape=jax.ShapeDtypeStruct((_N, _D), jnp.float32),
    )(h, w, b)


def _node_upd_body(hn_ref, p1_ref, p2_ref, p3_ref, p4_ref, w_ref, b_ref, o_ref):
    h = (
        hn_ref[...]
        + (p1_ref[...] + p2_ref[...])
        + (p3_ref[...] + p4_ref[...])
    )
    o_ref[...] = (
        jnp.dot(h, w_ref[...], preferred_element_type=jnp.float32) + b_ref[...]
    )


def _tc_node_upd(hn, p1, p2, p3, p4, w, b):
    return pl.pallas_call(
        _node_upd_body,
        grid=(_N // _RB,),
        in_specs=[
            pl.BlockSpec((_RB, _D), lambda i: (i, 0)),
            pl.BlockSpec((_RB, _D), lambda i: (i, 0)),
            pl.BlockSpec((_RB, _D), lambda i: (i, 0)),
            pl.BlockSpec((_RB, _D), lambda i: (i, 0)),
            pl.BlockSpec((_RB, _D), lambda i: (i, 0)),
            pl.BlockSpec((_D, _D), lambda i: (0, 0)),
            pl.BlockSpec((1, _D), lambda i: (0, 0)),
        ],
        out_specs=pl.BlockSpec((_RB, _D), lambda i: (i, 0)),
        out_shape=jax.ShapeDtypeStruct((_N, _D), jnp.float32),
    )(hn, p1, p2, p3, p4, w, b)


def _edge_body(g_ref, ef_ref, we_ref, be_ref, wm_ref, bm_ref, o_ref):
    me = (
        jnp.dot(ef_ref[...], we_ref[...], preferred_element_type=jnp.float32)
        + be_ref[...]
    )
    t = g_ref[...] * me
    o_ref[...] = jnp.tanh(
        jnp.dot(t, wm_ref[...], preferred_element_type=jnp.float32) + bm_ref[...]
    )


def _tc_edge(g, ef, we, be, wm, bm, ne):
    return pl.pallas_call(
        _edge_body,
        grid=(ne // _BE,),
        in_specs=[
            pl.BlockSpec((_BE, _D), lambda i: (i, 0)),
            pl.BlockSpec((_BE, _DE), lambda i: (i, 0)),
            pl.BlockSpec((_DE, _D), lambda i: (0, 0)),
            pl.BlockSpec((1, _D), lambda i: (0, 0)),
            pl.BlockSpec((_D, _D), lambda i: (0, 0)),
            pl.BlockSpec((1, _D), lambda i: (0, 0)),
        ],
        out_specs=pl.BlockSpec((_BE, _D), lambda i: (i, 0)),
        out_shape=jax.ShapeDtypeStruct((ne, _D), jnp.float32),
    )(g, ef, we, be, wm, bm)


def _final_body(hn_ref, p1_ref, p2_ref, p3_ref, p4_ref, wl_ref, bl_ref, o_ref):
    h = (
        hn_ref[...]
        + (p1_ref[...] + p2_ref[...])
        + (p3_ref[...] + p4_ref[...])
    )
    s = jnp.sum(h, axis=0, keepdims=True) * (1.0 / _N)
    o_ref[...] = (
        jnp.dot(s, wl_ref[...], preferred_element_type=jnp.float32) + bl_ref[...]
    )


def _tc_final(hn, p1, p2, p3, p4, wl, bl):
    return pl.pallas_call(
        _final_body,
        out_shape=jax.ShapeDtypeStruct((1, _EMB), jnp.float32),
    )(hn, p1, p2, p3, p4, wl, bl)


def _sc_gather(table, idx3d, ne):
    rpw = ne // _NW
    nch = rpw // _CH
    ngrp = nch // _GNBUF
    leftover = nch - ngrp * _GNBUF
    mesh = plsc.VectorSubcoreMesh(core_axis_name="c", subcore_axis_name="s")

    @functools.partial(
        pl.kernel,
        mesh=mesh,
        out_type=jax.ShapeDtypeStruct((ne, _D), jnp.float32),
        scratch_types=[
            pltpu.VMEM((nch, _CH), jnp.int32),
            pltpu.VMEM((_GNBUF, _CH, _D), jnp.float32),
            pltpu.VMEM_SHARED((_N, _D), jnp.float32),
            pltpu.SemaphoreType.DMA,
            pltpu.SemaphoreType.DMA,
        ],
    )
    def gather_kernel(table_hbm, idx_hbm, out_hbm, idx_v, rows_v, tab, gsem,
                      osem):
        s = lax.axis_index("s")
        wid = s * _NC + lax.axis_index("c")
        ebase = wid * rpw
        pltpu.sync_copy(idx_hbm.at[wid], idx_v)
        pltpu.sync_copy(
            table_hbm.at[pl.ds(s * _TPS, _TPS)], tab.at[pl.ds(s * _TPS, _TPS)]
        )

        @pl.when(s == _NS - 1)
        def _():
            pltpu.sync_copy(
                table_hbm.at[pl.ds(_NS * _TPS, _TAIL)],
                tab.at[pl.ds(_NS * _TPS, _TAIL)],
            )

        plsc.subcore_barrier()
        for b in range(_GNBUF):
            pltpu.async_copy(tab.at[idx_v.at[b]], rows_v.at[b], gsem)

        def outer(grp, carry):
            for b in range(_GNBUF):
                t = grp * _GNBUF + b
                pltpu.make_async_copy(
                    tab.at[idx_v.at[t]], rows_v.at[b], gsem
                ).wait()
                pltpu.async_copy(
                    rows_v.at[b], out_hbm.at[pl.ds(ebase + t * _CH, _CH)], osem
                )
            for b in range(_GNBUF):
                t = grp * _GNBUF + b
                pltpu.make_async_copy(
                    rows_v.at[b], out_hbm.at[pl.ds(ebase + t * _CH, _CH)], osem
                ).wait()

                @pl.when(grp + 1 < ngrp)
                def _():
                    t2 = (grp + 1) * _GNBUF + b
                    pltpu.async_copy(tab.at[idx_v.at[t2]], rows_v.at[b], gsem)
            return carry

        lax.fori_loop(0, ngrp, outer, 0)
        for t in range(ngrp * _GNBUF, ngrp * _GNBUF + leftover):
            pltpu.async_copy(tab.at[idx_v.at[t]], rows_v.at[0], gsem).wait()
            pltpu.sync_copy(rows_v.at[0], out_hbm.at[pl.ds(ebase + t * _CH, _CH)])

    return gather_kernel(table, idx3d)


def _sc_scatter(m, idx3d, zinit, ne):
    rpw = ne // _NW
    snch = rpw // _SCH
    sngrp = snch // _SNBUF
    sleft = snch - sngrp * _SNBUF
    mesh = plsc.VectorSubcoreMesh(core_axis_name="c", subcore_axis_name="s")

    @functools.partial(
        pl.kernel,
        mesh=mesh,
        out_type=jax.ShapeDtypeStruct((_NC * _N, _D), jnp.float32),
        scratch_types=[
            pltpu.VMEM((snch, _SCH), jnp.int32),
            pltpu.VMEM((_SNBUF, _SCH, _D), jnp.float32),
            pltpu.VMEM_SHARED((_N, _D), jnp.float32),
            pltpu.SemaphoreType.DMA,
            pltpu.SemaphoreType.DMA,
        ],
    )
    def scatter_kernel(m_hbm, idx_hbm, z_hbm, out_hbm, idx_v, rows_v, acc, sem,
                       ssem):
        c = lax.axis_index("c")
        s = lax.axis_index("s")
        wid = s * _NC + c
        ebase = wid * rpw
        pltpu.sync_copy(idx_hbm.at[wid], idx_v)
        pltpu.sync_copy(z_hbm.at[pl.ds(0, _TPS)], acc.at[pl.ds(s * _TPS, _TPS)])

        @pl.when(s == _NS - 1)
        def _():
            pltpu.sync_copy(
                z_hbm.at[pl.ds(0, _TAIL)], acc.at[pl.ds(_NS * _TPS, _TAIL)]
            )

        plsc.subcore_barrier()
        for b in range(2):
            pltpu.async_copy(
                m_hbm.at[pl.ds(ebase + b * _SCH, _SCH)], rows_v.at[b], sem
            )

        def step(u, b):
            pltpu.make_async_copy(
                m_hbm.at[pl.ds(ebase + u * _SCH, _SCH)], rows_v.at[b], sem
            ).wait()
            pltpu.async_copy(rows_v.at[b], acc.at[idx_v.at[u]], ssem, add=True)

            @pl.when(u >= 1)
            def _():
                bp = (b + 2) % _SNBUF
                pltpu.make_async_copy(
                    rows_v.at[bp], acc.at[idx_v.at[u - 1]], ssem
                ).wait()

            @pl.when(u + 2 < snch)
            def _():
                pltpu.async_copy(
                    m_hbm.at[pl.ds(ebase + (u + 2) * _SCH, _SCH)],
                    rows_v.at[(b + 2) % _SNBUF], sem,
                )

        def outer(grp, carry):
            for b in range(_SNBUF):
                step(grp * _SNBUF + b, b)
            return carry

        lax.fori_loop(0, sngrp, outer, 0)
        for t in range(sngrp * _SNBUF, sngrp * _SNBUF + sleft):
            step(jnp.int32(t), t % _SNBUF)
        lastu = sngrp * _SNBUF + sleft - 1
        pltpu.make_async_copy(
            rows_v.at[lastu % _SNBUF], acc.at[idx_v.at[lastu]], ssem
        ).wait()
        plsc.subcore_barrier()
        pltpu.sync_copy(
            acc.at[pl.ds(s * _TPS, _TPS)],
            out_hbm.at[pl.ds(c * _N + s * _TPS, _TPS)],
        )

        @pl.when(s == _NS - 1)
        def _():
            pltpu.sync_copy(
                acc.at[pl.ds(_NS * _TPS, _TAIL)],
                out_hbm.at[pl.ds(c * _N + _NS * _TPS, _TAIL)],
            )

    return scatter_kernel(m, idx3d, zinit)


_EA = 192000
_EB = _E - _EA


def kernel(x, edge_index, edge_features, Wn, bn, We, be, Wm, bm, Wlin, blin):
    src = edge_index[0].astype(jnp.int32)
    dst = edge_index[1].astype(jnp.int32)
    srcA = src[:_EA].reshape(_NW, _EA // _NW // _CH, _CH)
    srcB = src[_EA:].reshape(_NW, _EB // _NW // _CH, _CH)
    dstA = dst[:_EA].reshape(_NW, _EA // _NW // _SCH, _SCH)
    dstB = dst[_EA:].reshape(_NW, _EB // _NW // _SCH, _SCH)
    efA = edge_features[:_EA]
    efB = edge_features[_EA:]
    zinit = jnp.zeros((_TPS + _TAIL, _D), jnp.float32)

    hn = _tc_node0(x, Wn[0], bn[0].reshape(1, _D))
    pA = pB = None
    for i in range(_L):
        if i > 0:
            hn = _tc_node_upd(
                hn, pA[:_N], pA[_N:], pB[:_N], pB[_N:], Wn[i],
                bn[i].reshape(1, _D),
            )
        be_i = be[i].reshape(1, _D)
        bm_i = bm[i].reshape(1, _D)
        gA = _sc_gather(hn, srcA, _EA)
        gB = _sc_gather(hn, srcB, _EB)
        mA = _tc_edge(gA, efA, We[i], be_i, Wm[i], bm_i, _EA)
        mB = _tc_edge(gB, efB, We[i], be_i, Wm[i], bm_i, _EB)
        pA = _sc_scatter(mA, dstA, zinit, _EA)
        pB = _sc_scatter(mB, dstB, zinit, _EB)

    out = _tc_final(
        hn, pA[:_N], pA[_N:], pB[:_N], pB[_N:], Wlin, blin.reshape(1, _EMB)
    )
    return out.reshape(_EMB)


# speedup vs baseline: 4.0139x; 1.0035x over previous
import functools

import jax
import jax.numpy as jnp
from jax import lax
from jax.experimental import pallas as pl
from jax.experimental.pallas import tpu as pltpu
from jax.experimental.pallas import tpu_sc as plsc

_N = 10000
_E = 320000
_D = 128
_DE = 16
_L = 4
_EMB = 128

_NC = 2
_NS = 16
_NW = _NC * _NS
_CH = 80
_RPW = _E // _NW
_GNBUF = 3
_TPS = 624
_TAIL = _N - _NS * _TPS

_RB = 2000
_BE = 2000


_SCH = 80
_SNBUF = 3


def _node0_body(h_ref, w_ref, b_ref, o_ref):
    o_ref[...] = (
        jnp.dot(h_ref[...], w_ref[...], preferred_element_type=jnp.float32)
        + b_ref[...]
    )


def _tc_node0(h, w, b):
    return pl.pallas_call(
        _node0_body,
        grid=(_N // _RB,),
        in_specs=[
            pl.BlockSpec((_RB, _D), lambda i: (i, 0)),
            pl.BlockSpec((_D, _D), lambda i: (0, 0)),
            pl.BlockSpec((1, _D), lambda i: (0, 0)),
        ],
        out_specs=pl.BlockSpec((_RB, _D), lambda i: (i, 0)),
        out_shape=jax.ShapeDtypeStruct((_N, _D), jnp.float32),
    )(h, w, b)


def _node_upd_body(hn_ref, p1_ref, p2_ref, p3_ref, p4_ref, w_ref, b_ref, o_ref):
    h = (
        hn_ref[...]
        + (p1_ref[...] + p2_ref[...])
        + (p3_ref[...] + p4_ref[...])
    )
    o_ref[...] = (
        jnp.dot(h, w_ref[...], preferred_element_type=jnp.float32) + b_ref[...]
    )


def _tc_node_upd(hn, p1, p2, p3, p4, w, b):
    return pl.pallas_call(
        _node_upd_body,
        grid=(_N // _RB,),
        in_specs=[
            pl.BlockSpec((_RB, _D), lambda i: (i, 0)),
            pl.BlockSpec((_RB, _D), lambda i: (i, 0)),
            pl.BlockSpec((_RB, _D), lambda i: (i, 0)),
            pl.BlockSpec((_RB, _D), lambda i: (i, 0)),
            pl.BlockSpec((_RB, _D), lambda i: (i, 0)),
            pl.BlockSpec((_D, _D), lambda i: (0, 0)),
            pl.BlockSpec((1, _D), lambda i: (0, 0)),
        ],
        out_specs=pl.BlockSpec((_RB, _D), lambda i: (i, 0)),
        out_shape=jax.ShapeDtypeStruct((_N, _D), jnp.float32),
    )(hn, p1, p2, p3, p4, w, b)


def _edge_body(g_ref, ef_ref, we_ref, be_ref, wm_ref, bm_ref, o_ref):
    me = (
        jnp.dot(ef_ref[...], we_ref[...], preferred_element_type=jnp.float32)
        + be_ref[...]
    )
    t = g_ref[...] * me
    o_ref[...] = jnp.tanh(
        jnp.dot(t, wm_ref[...], preferred_element_type=jnp.float32) + bm_ref[...]
    )


def _tc_edge(g, ef, we, be, wm, bm, ne):
    return pl.pallas_call(
        _edge_body,
        grid=(ne // _BE,),
        in_specs=[
            pl.BlockSpec((_BE, _D), lambda i: (i, 0)),
            pl.BlockSpec((_BE, _DE), lambda i: (i, 0)),
            pl.BlockSpec((_DE, _D), lambda i: (0, 0)),
            pl.BlockSpec((1, _D), lambda i: (0, 0)),
            pl.BlockSpec((_D, _D), lambda i: (0, 0)),
            pl.BlockSpec((1, _D), lambda i: (0, 0)),
        ],
        out_specs=pl.BlockSpec((_BE, _D), lambda i: (i, 0)),
        out_shape=jax.ShapeDtypeStruct((ne, _D), jnp.float32),
    )(g, ef, we, be, wm, bm)


def _final_body(hn_ref, p1_ref, p2_ref, p3_ref, p4_ref, wl_ref, bl_ref, o_ref):
    h = (
        hn_ref[...]
        + (p1_ref[...] + p2_ref[...])
        + (p3_ref[...] + p4_ref[...])
    )
    s = jnp.sum(h, axis=0, keepdims=True) * (1.0 / _N)
    o_ref[...] = (
        jnp.dot(s, wl_ref[...], preferred_element_type=jnp.float32) + bl_ref[...]
    )


def _tc_final(hn, p1, p2, p3, p4, wl, bl):
    return pl.pallas_call(
        _final_body,
        out_shape=jax.ShapeDtypeStruct((1, _EMB), jnp.float32),
    )(hn, p1, p2, p3, p4, wl, bl)


def _sc_gather(table, idx3d, ne):
    rpw = ne // _NW
    nch = rpw // _CH
    ngrp = nch // _GNBUF
    leftover = nch - ngrp * _GNBUF
    mesh = plsc.VectorSubcoreMesh(core_axis_name="c", subcore_axis_name="s")

    @functools.partial(
        pl.kernel,
        mesh=mesh,
        out_type=jax.ShapeDtypeStruct((ne, _D), jnp.float32),
        scratch_types=[
            pltpu.VMEM((nch, _CH), jnp.int32),
            pltpu.VMEM((_GNBUF, _CH, _D), jnp.float32),
            pltpu.VMEM_SHARED((_N, _D), jnp.float32),
            pltpu.SemaphoreType.DMA,
            pltpu.SemaphoreType.DMA,
        ],
    )
    def gather_kernel(table_hbm, idx_hbm, out_hbm, idx_v, rows_v, tab, gsem,
                      osem):
        s = lax.axis_index("s")
        wid = s * _NC + lax.axis_index("c")
        ebase = wid * rpw
        pltpu.sync_copy(idx_hbm.at[wid], idx_v)
        pltpu.sync_copy(
            table_hbm.at[pl.ds(s * _TPS, _TPS)], tab.at[pl.ds(s * _TPS, _TPS)]
        )

        @pl.when(s == _NS - 1)
        def _():
            pltpu.sync_copy(
                table_hbm.at[pl.ds(_NS * _TPS, _TAIL)],
                tab.at[pl.ds(_NS * _TPS, _TAIL)],
            )

        plsc.subcore_barrier()
        for b in range(2):
            pltpu.async_copy(tab.at[idx_v.at[b]], rows_v.at[b], gsem)

        def step(u, b):
            pltpu.make_async_copy(tab.at[idx_v.at[u]], rows_v.at[b], gsem).wait()
            pltpu.async_copy(
                rows_v.at[b], out_hbm.at[pl.ds(ebase + u * _CH, _CH)], osem
            )

            @pl.when(u >= 1)
            def _():
                bp = (b + 2) % _GNBUF
                pltpu.make_async_copy(
                    rows_v.at[bp],
                    out_hbm.at[pl.ds(ebase + (u - 1) * _CH, _CH)], osem,
                ).wait()

            @pl.when(u + 2 < nch)
            def _():
                pltpu.async_copy(
                    tab.at[idx_v.at[u + 2]], rows_v.at[(b + 2) % _GNBUF], gsem
                )

        def outer(grp, carry):
            for b in range(_GNBUF):
                step(grp * _GNBUF + b, b)
            return carry

        lax.fori_loop(0, ngrp, outer, 0)
        for t in range(ngrp * _GNBUF, ngrp * _GNBUF + leftover):
            step(jnp.int32(t), t % _GNBUF)
        lastu = ngrp * _GNBUF + leftover - 1
        pltpu.make_async_copy(
            rows_v.at[lastu % _GNBUF],
            out_hbm.at[pl.ds(ebase + lastu * _CH, _CH)], osem,
        ).wait()

    return gather_kernel(table, idx3d)


def _sc_scatter(m, idx3d, zinit, ne):
    rpw = ne // _NW
    snch = rpw // _SCH
    sngrp = snch // _SNBUF
    sleft = snch - sngrp * _SNBUF
    mesh = plsc.VectorSubcoreMesh(core_axis_name="c", subcore_axis_name="s")

    @functools.partial(
        pl.kernel,
        mesh=mesh,
        out_type=jax.ShapeDtypeStruct((_NC * _N, _D), jnp.float32),
        scratch_types=[
            pltpu.VMEM((snch, _SCH), jnp.int32),
            pltpu.VMEM((_SNBUF, _SCH, _D), jnp.float32),
            pltpu.VMEM_SHARED((_N, _D), jnp.float32),
            pltpu.SemaphoreType.DMA,
            pltpu.SemaphoreType.DMA,
        ],
    )
    def scatter_kernel(m_hbm, idx_hbm, z_hbm, out_hbm, idx_v, rows_v, acc, sem,
                       ssem):
        c = lax.axis_index("c")
        s = lax.axis_index("s")
        wid = s * _NC + c
        ebase = wid * rpw
        pltpu.sync_copy(idx_hbm.at[wid], idx_v)
        pltpu.sync_copy(z_hbm.at[pl.ds(0, _TPS)], acc.at[pl.ds(s * _TPS, _TPS)])

        @pl.when(s == _NS - 1)
        def _():
            pltpu.sync_copy(
                z_hbm.at[pl.ds(0, _TAIL)], acc.at[pl.ds(_NS * _TPS, _TAIL)]
            )

        plsc.subcore_barrier()
        for b in range(2):
            pltpu.async_copy(
                m_hbm.at[pl.ds(ebase + b * _SCH, _SCH)], rows_v.at[b], sem
            )

        def step(u, b):
            pltpu.make_async_copy(
                m_hbm.at[pl.ds(ebase + u * _SCH, _SCH)], rows_v.at[b], sem
            ).wait()
            pltpu.async_copy(rows_v.at[b], acc.at[idx_v.at[u]], ssem, add=True)

            @pl.when(u >= 1)
            def _():
                bp = (b + 2) % _SNBUF
                pltpu.make_async_copy(
                    rows_v.at[bp], acc.at[idx_v.at[u - 1]], ssem
                ).wait()

            @pl.when(u + 2 < snch)
            def _():
                pltpu.async_copy(
                    m_hbm.at[pl.ds(ebase + (u + 2) * _SCH, _SCH)],
                    rows_v.at[(b + 2) % _SNBUF], sem,
                )

        def outer(grp, carry):
            for b in range(_SNBUF):
                step(grp * _SNBUF + b, b)
            return carry

        lax.fori_loop(0, sngrp, outer, 0)
        for t in range(sngrp * _SNBUF, sngrp * _SNBUF + sleft):
            step(jnp.int32(t), t % _SNBUF)
        lastu = sngrp * _SNBUF + sleft - 1
        pltpu.make_async_copy(
            rows_v.at[lastu % _SNBUF], acc.at[idx_v.at[lastu]], ssem
        ).wait()
        plsc.subcore_barrier()
        pltpu.sync_copy(
            acc.at[pl.ds(s * _TPS, _TPS)],
            out_hbm.at[pl.ds(c * _N + s * _TPS, _TPS)],
        )

        @pl.when(s == _NS - 1)
        def _():
            pltpu.sync_copy(
                acc.at[pl.ds(_NS * _TPS, _TAIL)],
                out_hbm.at[pl.ds(c * _N + _NS * _TPS, _TAIL)],
            )

    return scatter_kernel(m, idx3d, zinit)


_EA = 192000
_EB = _E - _EA


def kernel(x, edge_index, edge_features, Wn, bn, We, be, Wm, bm, Wlin, blin):
    src = edge_index[0].astype(jnp.int32)
    dst = edge_index[1].astype(jnp.int32)
    srcA = src[:_EA].reshape(_NW, _EA // _NW // _CH, _CH)
    srcB = src[_EA:].reshape(_NW, _EB // _NW // _CH, _CH)
    dstA = dst[:_EA].reshape(_NW, _EA // _NW // _SCH, _SCH)
    dstB = dst[_EA:].reshape(_NW, _EB // _NW // _SCH, _SCH)
    efA = edge_features[:_EA]
    efB = edge_features[_EA:]
    zinit = jnp.zeros((_TPS + _TAIL, _D), jnp.float32)

    hn = _tc_node0(x, Wn[0], bn[0].reshape(1, _D))
    pA = pB = None
    for i in range(_L):
        if i > 0:
            hn = _tc_node_upd(
                hn, pA[:_N], pA[_N:], pB[:_N], pB[_N:], Wn[i],
                bn[i].reshape(1, _D),
            )
        be_i = be[i].reshape(1, _D)
        bm_i = bm[i].reshape(1, _D)
        gA = _sc_gather(hn, srcA, _EA)
        gB = _sc_gather(hn, srcB, _EB)
        mA = _tc_edge(gA, efA, We[i], be_i, Wm[i], bm_i, _EA)
        mB = _tc_edge(gB, efB, We[i], be_i, Wm[i], bm_i, _EB)
        pA = _sc_scatter(mA, dstA, zinit, _EA)
        pB = _sc_scatter(mB, dstB, zinit, _EB)

    out = _tc_final(
        hn, pA[:_N], pA[_N:], pB[:_N], pB[_N:], Wlin, blin.reshape(1, _EMB)
    )
    return out.reshape(_EMB)
